# Initial kernel scaffold; baseline (speedup 1.0000x reference)
import jax
import jax.numpy as jnp
from jax.experimental import pallas as pl


def kernel(feats, node2edge_idx, edge_emb, edge_node_adj, id_emb, W_prep, W_edge_prep, W_e_self_0, W_e_neigh_0, W_n_self_0, W_n_neigh_0, W_e_self_1, W_e_neigh_1, W_n_self_1, W_n_neigh_1):
    raise NotImplementedError("write your pallas kernel here")


# speedup vs baseline: 5.9268x; 5.9268x over previous
import functools

import jax
import jax.numpy as jnp
from jax import lax
from jax.experimental import pallas as pl
from jax.experimental.pallas import tpu as pltpu
from jax.experimental.pallas import tpu_sc as plsc

N = 10000
S = 32
E = 320000
D_FEAT = 128
D_EDGE = 16
P = 64
OUT = 64

NC = 2
NS = 16
NW = NC * NS
L = 16

EC = 128
N_ECHUNK = E // EC
KN = 4
N_NCHUNK = N // KN

_mesh = plsc.VectorSubcoreMesh(core_axis_name="c", subcore_axis_name="s")
_sc_params = pltpu.CompilerParams(use_tc_tiling_on_sc=False)


def _worker_id():
    return lax.axis_index("s") * NC + lax.axis_index("c")


@functools.partial(
    pl.kernel,
    mesh=_mesh,
    out_type=jax.ShapeDtypeStruct((E, OUT), jnp.float32),
    scratch_types=[
        pltpu.VMEM((EC,), jnp.int32),
        pltpu.VMEM((EC,), jnp.int32),
        pltpu.VMEM((EC, OUT), jnp.float32),
        pltpu.VMEM((EC, OUT), jnp.float32),
        pltpu.VMEM((EC, OUT), jnp.float32),
        pltpu.VMEM((EC, OUT), jnp.float32),
        pltpu.SemaphoreType.DMA,
    ],
    compiler_params=_sc_params,
)
def _sc_edge_stage(g_hbm, a0_hbm, a1_hbm, t_hbm, ne0_hbm,
                   i0_v, i1_v, r0_v, r1_v, t_v, o_v, sem):
    wid = _worker_id()
    n_iter = (N_ECHUNK - wid + NW - 1) // NW

    def chunk_body(jj, _):
        jg = wid + jj * NW
        e0 = jg * EC
        pltpu.sync_copy(a0_hbm.at[pl.ds(e0, EC)], i0_v)
        pltpu.sync_copy(a1_hbm.at[pl.ds(e0, EC)], i1_v)
        c0 = pltpu.async_copy(g_hbm.at[i0_v], r0_v, sem)
        c1 = pltpu.async_copy(g_hbm.at[i1_v], r1_v, sem)
        c2 = pltpu.async_copy(t_hbm.at[pl.ds(e0, EC)], t_v, sem)
        c0.wait()
        c1.wait()
        c2.wait()

        def edge_body(c, _):
            for k in range(OUT // L):
                sl = pl.ds(k * L, L)
                v = t_v[c, sl] + r0_v[c, sl] + r1_v[c, sl]
                o_v[c, sl] = jnp.maximum(v, 0.0)
            return _

        lax.fori_loop(0, EC, edge_body, None)
        pltpu.sync_copy(o_v, ne0_hbm.at[pl.ds(e0, EC)])
        return _

    lax.fori_loop(0, n_iter, chunk_body, None)


@functools.partial(
    pl.kernel,
    mesh=_mesh,
    out_type=(
        jax.ShapeDtypeStruct((N, D_EDGE), jnp.float32),
        jax.ShapeDtypeStruct((N, OUT), jnp.float32),
    ),
    scratch_types=[
        pltpu.VMEM((KN * S,), jnp.int32),
        pltpu.VMEM((KN * S, D_EDGE), jnp.float32),
        pltpu.VMEM((KN * S, OUT), jnp.float32),
        pltpu.VMEM((KN, D_EDGE), jnp.float32),
        pltpu.VMEM((KN, OUT), jnp.float32),
        pltpu.SemaphoreType.DMA,
    ],
    compiler_params=_sc_params,
)
def _sc_node_stage(n2e_hbm, ee_hbm, ne0_hbm, em_hbm, me_hbm,
                   idx_v, er_v, nr_v, em_v, me_v, sem):
    wid = _worker_id()
    n_iter = (N_NCHUNK - wid + NW - 1) // NW

    def chunk_body(jj, _):
        jg = wid + jj * NW
        n0 = jg * KN
        pltpu.sync_copy(n2e_hbm.at[pl.ds(n0 * S, KN * S)], idx_v)
        c0 = pltpu.async_copy(ee_hbm.at[idx_v], er_v, sem)
        c1 = pltpu.async_copy(ne0_hbm.at[idx_v], nr_v, sem)
        c0.wait()
        c1.wait()
        for jn in range(KN):
            def acc_body(s, accs):
                base = jn * S + s
                new = tuple(
                    accs[k] + nr_v[base, pl.ds(k * L, L)]
                    for k in range(OUT // L)
                ) + (accs[OUT // L] + er_v[base, pl.ds(0, D_EDGE)],)
                return new

            zero64 = tuple(jnp.zeros((L,), jnp.float32) for _ in range(OUT // L))
            zero16 = (jnp.zeros((D_EDGE,), jnp.float32),)
            accs = lax.fori_loop(0, S, acc_body, zero64 + zero16)
            for k in range(OUT // L):
                me_v[jn, pl.ds(k * L, L)] = accs[k]
            em_v[jn, pl.ds(0, D_EDGE)] = accs[OUT // L]
        pltpu.sync_copy(em_v, em_hbm.at[pl.ds(n0, KN)])
        pltpu.sync_copy(me_v, me_hbm.at[pl.ds(n0, KN)])
        return _

    lax.fori_loop(0, n_iter, chunk_body, None)


def _prep_body(feats_ref, wp_ref, wen0_ref, wep_ref, wes0_ref, wnn0_ref,
               g_ref, wces_ref, wcn0_ref):
    h = jnp.dot(feats_ref[...], wp_ref[...], preferred_element_type=jnp.float32)
    g_ref[...] = 0.5 * jnp.dot(h, wen0_ref[...], preferred_element_type=jnp.float32)
    wces_ref[...] = jnp.dot(wep_ref[...], wes0_ref[...], preferred_element_type=jnp.float32)
    wcn0_ref[...] = jnp.dot(wep_ref[...], wnn0_ref[...], preferred_element_type=jnp.float32)


def _t_body(ee_ref, w_ref, t_ref):
    t_ref[...] = jnp.dot(ee_ref[...], w_ref[...], preferred_element_type=jnp.float32)


def _final_body(id_ref, em_ref, me_ref, wns0_ref, wcn0_ref, wns1_ref, wnn1_ref,
                o_ref):
    inv_s = 1.0 / S
    m0 = jnp.dot(em_ref[...] * inv_s, wcn0_ref[...], preferred_element_type=jnp.float32)
    nf0 = jnp.maximum(jnp.dot(id_ref[...], wns0_ref[...], preferred_element_type=jnp.float32) + m0, 0.0)
    m1 = jnp.dot(me_ref[...] * inv_s, wnn1_ref[...], preferred_element_type=jnp.float32)
    nf1 = jnp.maximum(jnp.dot(nf0, wns1_ref[...], preferred_element_type=jnp.float32) + m1, 0.0)
    o_ref[...] = jnp.concatenate([nf0, nf1], axis=-1)


_TBLOCK = 3200


def kernel(feats, node2edge_idx, edge_emb, edge_node_adj, id_emb,
           W_prep, W_edge_prep,
           W_e_self_0, W_e_neigh_0, W_n_self_0, W_n_neigh_0,
           W_e_self_1, W_e_neigh_1, W_n_self_1, W_n_neigh_1):
    g, wces, wcn0 = pl.pallas_call(
        _prep_body,
        out_shape=[
            jax.ShapeDtypeStruct((N, P), jnp.float32),
            jax.ShapeDtypeStruct((D_EDGE, OUT), jnp.float32),
            jax.ShapeDtypeStruct((D_EDGE, OUT), jnp.float32),
        ],
    )(feats, W_prep, W_e_neigh_0, W_edge_prep, W_e_self_0, W_n_neigh_0)

    t = pl.pallas_call(
        _t_body,
        grid=(E // _TBLOCK,),
        in_specs=[
            pl.BlockSpec((_TBLOCK, D_EDGE), lambda i: (i, 0)),
            pl.BlockSpec((D_EDGE, OUT), lambda i: (0, 0)),
        ],
        out_specs=pl.BlockSpec((_TBLOCK, OUT), lambda i: (i, 0)),
        out_shape=jax.ShapeDtypeStruct((E, OUT), jnp.float32),
    )(edge_emb, wces)

    a0 = jnp.reshape(edge_node_adj[:, 0], (E,))
    a1 = jnp.reshape(edge_node_adj[:, 1], (E,))

    ne0 = _sc_edge_stage(g, a0, a1, t)

    n2e_flat = jnp.reshape(node2edge_idx, (N * S,))
    em_sum, me_sum = _sc_node_stage(n2e_flat, edge_emb, ne0)

    out = pl.pallas_call(
        _final_body,
        out_shape=jax.ShapeDtypeStruct((N, 2 * OUT), jnp.float32),
    )(id_emb, em_sum, me_sum, W_n_self_0, wcn0, W_n_self_1, W_n_neigh_1)

    return out[None]


# speedup vs baseline: 6.7151x; 1.1330x over previous
import functools

import jax
import jax.numpy as jnp
from jax import lax
from jax.experimental import pallas as pl
from jax.experimental.pallas import tpu as pltpu
from jax.experimental.pallas import tpu_sc as plsc

N = 10000
S = 32
E = 320000
D_FEAT = 128
D_EDGE = 16
P = 64
OUT = 64

NC = 2
NS = 16
NW = NC * NS
L = 16

EC = 128
N_ECHUNK = E // EC
KN = 4
N_NCHUNK = N // KN
BASE_CH = N_ECHUNK // NW
EXTRA = N_ECHUNK - BASE_CH * NW
MAXC = BASE_CH + 1

_mesh = plsc.VectorSubcoreMesh(core_axis_name="c", subcore_axis_name="s")
_sc_params = pltpu.CompilerParams(use_tc_tiling_on_sc=False)


def _worker_id():
    return lax.axis_index("s") * NC + lax.axis_index("c")


def _chunk_range():
    wid = _worker_id()
    base_chunk = wid * BASE_CH + jnp.minimum(wid, EXTRA)
    n_iter = BASE_CH + jnp.where(wid < EXTRA, 1, 0)
    return wid, base_chunk, n_iter


@functools.partial(
    pl.kernel,
    mesh=_mesh,
    out_type=jax.ShapeDtypeStruct((E, OUT), jnp.float32),
    scratch_types=[
        pltpu.VMEM((MAXC * EC,), jnp.int32),
        pltpu.VMEM((MAXC * EC,), jnp.int32),
        pltpu.VMEM((2, EC, OUT), jnp.float32),
        pltpu.VMEM((2, EC, OUT), jnp.float32),
        pltpu.VMEM((2, EC, OUT), jnp.float32),
        pltpu.SemaphoreType.DMA,
        pltpu.SemaphoreType.DMA,
    ],
    compiler_params=_sc_params,
)
def _sc_edge_stage(g_hbm, a0_hbm, a1_hbm, s_hbm,
                   i0_v, i1_v, r0_v, r1_v, o_v, sem_g, sem_o):
    wid, base_chunk, n_iter = _chunk_range()
    e_start = base_chunk * EC

    pltpu.sync_copy(a0_hbm.at[pl.ds(e_start, BASE_CH * EC)],
                    i0_v.at[pl.ds(0, BASE_CH * EC)])
    pltpu.sync_copy(a1_hbm.at[pl.ds(e_start, BASE_CH * EC)],
                    i1_v.at[pl.ds(0, BASE_CH * EC)])

    @pl.when(wid < EXTRA)
    def _():
        pltpu.sync_copy(a0_hbm.at[pl.ds(e_start + BASE_CH * EC, EC)],
                        i0_v.at[pl.ds(BASE_CH * EC, EC)])
        pltpu.sync_copy(a1_hbm.at[pl.ds(e_start + BASE_CH * EC, EC)],
                        i1_v.at[pl.ds(BASE_CH * EC, EC)])

    def issue(c, slot):
        pltpu.async_copy(g_hbm.at[i0_v.at[pl.ds(c * EC, EC)]], r0_v.at[slot], sem_g)
        pltpu.async_copy(g_hbm.at[i1_v.at[pl.ds(c * EC, EC)]], r1_v.at[slot], sem_g)

    def drain_gathers(slot):
        pltpu.make_async_copy(g_hbm.at[i0_v.at[pl.ds(0, EC)]], r0_v.at[slot], sem_g).wait()
        pltpu.make_async_copy(g_hbm.at[i1_v.at[pl.ds(0, EC)]], r1_v.at[slot], sem_g).wait()

    def compute(slot):
        def body(c2, _):
            for u in range(2):
                c = c2 * 2 + u
                for k in range(OUT // L):
                    sl = pl.ds(k * L, L)
                    o_v[slot, c, sl] = r0_v[slot, c, sl] + r1_v[slot, c, sl]
            return _
        lax.fori_loop(0, EC // 2, body, None)

    def fire_store(c, slot):
        pltpu.async_copy(o_v.at[slot], s_hbm.at[pl.ds((base_chunk + c) * EC, EC)], sem_o)

    def drain_store(slot):
        pltpu.make_async_copy(o_v.at[slot], s_hbm.at[pl.ds(0, EC)], sem_o).wait()

    issue(0, 0)
    issue(1, 1)

    def pair_body(p, _):
        for slot in range(2):
            c = 2 * p + slot

            @pl.when(c >= 2)
            def _():
                drain_store(slot)

            drain_gathers(slot)
            compute(slot)
            fire_store(c, slot)

            @pl.when(c + 2 < n_iter)
            def _():
                issue(c + 2, slot)
        return _

    lax.fori_loop(0, BASE_CH // 2, pair_body, None)

    @pl.when(n_iter > BASE_CH)
    def _():
        drain_store(0)
        drain_gathers(0)
        compute(0)
        fire_store(BASE_CH, 0)

    drain_store(1)
    drain_store(0)


@functools.partial(
    pl.kernel,
    mesh=_mesh,
    out_type=(
        jax.ShapeDtypeStruct((N, D_EDGE), jnp.float32),
        jax.ShapeDtypeStruct((N, OUT), jnp.float32),
    ),
    scratch_types=[
        pltpu.VMEM((MAXC * KN * S,), jnp.int32),
        pltpu.VMEM((2, KN * S, D_EDGE), jnp.float32),
        pltpu.VMEM((2, KN * S, OUT), jnp.float32),
        pltpu.VMEM((2, KN, D_EDGE), jnp.float32),
        pltpu.VMEM((2, KN, OUT), jnp.float32),
        pltpu.SemaphoreType.DMA,
        pltpu.SemaphoreType.DMA,
    ],
    compiler_params=_sc_params,
)
def _sc_node_stage(n2e_hbm, ee_hbm, ne0_hbm, em_hbm, me_hbm,
                   idx_v, er_v, nr_v, em_v, me_v, sem_g, sem_o):
    wid, base_chunk, n_iter = _chunk_range()
    i_start = base_chunk * KN * S
    CL = KN * S

    pltpu.sync_copy(n2e_hbm.at[pl.ds(i_start, BASE_CH * CL)],
                    idx_v.at[pl.ds(0, BASE_CH * CL)])

    @pl.when(wid < EXTRA)
    def _():
        pltpu.sync_copy(n2e_hbm.at[pl.ds(i_start + BASE_CH * CL, CL)],
                        idx_v.at[pl.ds(BASE_CH * CL, CL)])

    def issue(c, slot):
        pltpu.async_copy(ee_hbm.at[idx_v.at[pl.ds(c * CL, CL)]], er_v.at[slot], sem_g)
        pltpu.async_copy(ne0_hbm.at[idx_v.at[pl.ds(c * CL, CL)]], nr_v.at[slot], sem_g)

    def drain_gathers(slot):
        pltpu.make_async_copy(ee_hbm.at[idx_v.at[pl.ds(0, CL)]], er_v.at[slot], sem_g).wait()
        pltpu.make_async_copy(ne0_hbm.at[idx_v.at[pl.ds(0, CL)]], nr_v.at[slot], sem_g).wait()

    def compute(slot):
        for jn in range(KN):
            def acc_body(s, accs):
                base = jn * S + s
                new = tuple(
                    accs[k] + nr_v[slot, base, pl.ds(k * L, L)]
                    for k in range(OUT // L)
                ) + (accs[OUT // L] + er_v[slot, base, pl.ds(0, D_EDGE)],)
                return new

            zero64 = tuple(jnp.zeros((L,), jnp.float32) for _ in range(OUT // L))
            zero16 = (jnp.zeros((D_EDGE,), jnp.float32),)
            accs = lax.fori_loop(0, S, acc_body, zero64 + zero16)
            for k in range(OUT // L):
                me_v[slot, jn, pl.ds(k * L, L)] = accs[k]
            em_v[slot, jn, pl.ds(0, D_EDGE)] = accs[OUT // L]

    def fire_store(c, slot):
        n0 = (base_chunk + c) * KN
        pltpu.async_copy(em_v.at[slot], em_hbm.at[pl.ds(n0, KN)], sem_o)
        pltpu.async_copy(me_v.at[slot], me_hbm.at[pl.ds(n0, KN)], sem_o)

    def drain_store(slot):
        pltpu.make_async_copy(em_v.at[slot], em_hbm.at[pl.ds(0, KN)], sem_o).wait()
        pltpu.make_async_copy(me_v.at[slot], me_hbm.at[pl.ds(0, KN)], sem_o).wait()

    issue(0, 0)
    issue(1, 1)

    def pair_body(p, _):
        for slot in range(2):
            c = 2 * p + slot

            @pl.when(c >= 2)
            def _():
                drain_store(slot)

            drain_gathers(slot)
            compute(slot)
            fire_store(c, slot)

            @pl.when(c + 2 < n_iter)
            def _():
                issue(c + 2, slot)
        return _

    lax.fori_loop(0, BASE_CH // 2, pair_body, None)

    @pl.when(n_iter > BASE_CH)
    def _():
        drain_store(0)
        drain_gathers(0)
        compute(0)
        fire_store(BASE_CH, 0)

    drain_store(1)
    drain_store(0)


def _prep_body(feats_ref, wp_ref, wen0_ref, wep_ref, wes0_ref, wnn0_ref,
               g_ref, wces_ref, wcn0_ref):
    h = jnp.dot(feats_ref[...], wp_ref[...], preferred_element_type=jnp.float32)
    g_ref[...] = 0.5 * jnp.dot(h, wen0_ref[...], preferred_element_type=jnp.float32)
    wces_ref[...] = jnp.dot(wep_ref[...], wes0_ref[...], preferred_element_type=jnp.float32)
    wcn0_ref[...] = jnp.dot(wep_ref[...], wnn0_ref[...], preferred_element_type=jnp.float32)


def _edge_body(ee_ref, w_ref, s_ref, ne0_ref):
    t = jnp.dot(ee_ref[...], w_ref[...], preferred_element_type=jnp.float32)
    ne0_ref[...] = jnp.maximum(t + s_ref[...], 0.0)


def _final_body(id_ref, em_ref, me_ref, wns0_ref, wcn0_ref, wns1_ref, wnn1_ref,
                o_ref):
    inv_s = 1.0 / S
    m0 = jnp.dot(em_ref[...] * inv_s, wcn0_ref[...], preferred_element_type=jnp.float32)
    nf0 = jnp.maximum(jnp.dot(id_ref[...], wns0_ref[...], preferred_element_type=jnp.float32) + m0, 0.0)
    m1 = jnp.dot(me_ref[...] * inv_s, wnn1_ref[...], preferred_element_type=jnp.float32)
    nf1 = jnp.maximum(jnp.dot(nf0, wns1_ref[...], preferred_element_type=jnp.float32) + m1, 0.0)
    o_ref[...] = jnp.concatenate([nf0, nf1], axis=-1)


_TBLOCK = 3200


def kernel(feats, node2edge_idx, edge_emb, edge_node_adj, id_emb,
           W_prep, W_edge_prep,
           W_e_self_0, W_e_neigh_0, W_n_self_0, W_n_neigh_0,
           W_e_self_1, W_e_neigh_1, W_n_self_1, W_n_neigh_1):
    g, wces, wcn0 = pl.pallas_call(
        _prep_body,
        out_shape=[
            jax.ShapeDtypeStruct((N, P), jnp.float32),
            jax.ShapeDtypeStruct((D_EDGE, OUT), jnp.float32),
            jax.ShapeDtypeStruct((D_EDGE, OUT), jnp.float32),
        ],
    )(feats, W_prep, W_e_neigh_0, W_edge_prep, W_e_self_0, W_n_neigh_0)

    a0 = jnp.reshape(edge_node_adj[:, 0], (E,))
    a1 = jnp.reshape(edge_node_adj[:, 1], (E,))

    s = _sc_edge_stage(g, a0, a1)

    ne0 = pl.pallas_call(
        _edge_body,
        grid=(E // _TBLOCK,),
        in_specs=[
            pl.BlockSpec((_TBLOCK, D_EDGE), lambda i: (i, 0)),
            pl.BlockSpec((D_EDGE, OUT), lambda i: (0, 0)),
            pl.BlockSpec((_TBLOCK, OUT), lambda i: (i, 0)),
        ],
        out_specs=pl.BlockSpec((_TBLOCK, OUT), lambda i: (i, 0)),
        out_shape=jax.ShapeDtypeStruct((E, OUT), jnp.float32),
    )(edge_emb, wces, s)

    n2e_flat = jnp.reshape(node2edge_idx, (N * S,))
    em_sum, me_sum = _sc_node_stage(n2e_flat, edge_emb, ne0)

    out = pl.pallas_call(
        _final_body,
        out_shape=jax.ShapeDtypeStruct((N, 2 * OUT), jnp.float32),
    )(id_emb, em_sum, me_sum, W_n_self_0, wcn0, W_n_self_1, W_n_neigh_1)

    return out[None]


# speedup vs baseline: 7.9939x; 1.1904x over previous
import functools

import jax
import jax.numpy as jnp
from jax import lax
from jax.experimental import pallas as pl
from jax.experimental.pallas import tpu as pltpu
from jax.experimental.pallas import tpu_sc as plsc

N = 10000
S = 32
E = 320000
D_FEAT = 128
D_EDGE = 16
P = 64
OUT = 64

NC = 2
NS = 16
NW = NC * NS
L = 16

EC = 128
N_ECHUNK = E // EC
KN = 4
N_NCHUNK = N // KN
BASE_CH = N_ECHUNK // NW
EXTRA = N_ECHUNK - BASE_CH * NW
MAXC = BASE_CH + 1

_mesh = plsc.VectorSubcoreMesh(core_axis_name="c", subcore_axis_name="s")
_sc_params = pltpu.CompilerParams(use_tc_tiling_on_sc=False)


def _worker_id():
    return lax.axis_index("s") * NC + lax.axis_index("c")


def _chunk_range():
    wid = _worker_id()
    base_chunk = wid * BASE_CH + jnp.minimum(wid, EXTRA)
    n_iter = BASE_CH + jnp.where(wid < EXTRA, 1, 0)
    return wid, base_chunk, n_iter


@functools.partial(
    pl.kernel,
    mesh=_mesh,
    out_type=jax.ShapeDtypeStruct((E, OUT), jnp.float32),
    scratch_types=[
        pltpu.VMEM((MAXC * EC,), jnp.int32),
        pltpu.VMEM((MAXC * EC,), jnp.int32),
        pltpu.VMEM((2, EC, OUT), jnp.float32),
        pltpu.VMEM((2, EC, OUT), jnp.float32),
        pltpu.VMEM((2, EC, OUT), jnp.float32),
        pltpu.SemaphoreType.DMA,
        pltpu.SemaphoreType.DMA,
    ],
    compiler_params=_sc_params,
)
def _sc_edge_stage(g_hbm, a0_hbm, a1_hbm, s_hbm,
                   i0_v, i1_v, r0_v, r1_v, o_v, sem_g, sem_o):
    wid, base_chunk, n_iter = _chunk_range()
    e_start = base_chunk * EC

    pltpu.sync_copy(a0_hbm.at[pl.ds(e_start, BASE_CH * EC)],
                    i0_v.at[pl.ds(0, BASE_CH * EC)])
    pltpu.sync_copy(a1_hbm.at[pl.ds(e_start, BASE_CH * EC)],
                    i1_v.at[pl.ds(0, BASE_CH * EC)])

    @pl.when(wid < EXTRA)
    def _():
        pltpu.sync_copy(a0_hbm.at[pl.ds(e_start + BASE_CH * EC, EC)],
                        i0_v.at[pl.ds(BASE_CH * EC, EC)])
        pltpu.sync_copy(a1_hbm.at[pl.ds(e_start + BASE_CH * EC, EC)],
                        i1_v.at[pl.ds(BASE_CH * EC, EC)])

    def issue(c, slot):
        pltpu.async_copy(g_hbm.at[i0_v.at[pl.ds(c * EC, EC)]], r0_v.at[slot], sem_g)
        pltpu.async_copy(g_hbm.at[i1_v.at[pl.ds(c * EC, EC)]], r1_v.at[slot], sem_g)

    def drain_gathers(slot):
        pltpu.make_async_copy(g_hbm.at[i0_v.at[pl.ds(0, EC)]], r0_v.at[slot], sem_g).wait()
        pltpu.make_async_copy(g_hbm.at[i1_v.at[pl.ds(0, EC)]], r1_v.at[slot], sem_g).wait()

    def compute(slot):
        def body(c2, _):
            for u in range(2):
                c = c2 * 2 + u
                for k in range(OUT // L):
                    sl = pl.ds(k * L, L)
                    o_v[slot, c, sl] = r0_v[slot, c, sl] + r1_v[slot, c, sl]
            return _
        lax.fori_loop(0, EC // 2, body, None)

    def fire_store(c, slot):
        pltpu.async_copy(o_v.at[slot], s_hbm.at[pl.ds((base_chunk + c) * EC, EC)], sem_o)

    def drain_store(slot):
        pltpu.make_async_copy(o_v.at[slot], s_hbm.at[pl.ds(0, EC)], sem_o).wait()

    issue(0, 0)
    issue(1, 1)

    def pair_body(p, _):
        for slot in range(2):
            c = 2 * p + slot

            @pl.when(c >= 2)
            def _():
                drain_store(slot)

            drain_gathers(slot)
            compute(slot)
            fire_store(c, slot)

            @pl.when(c + 2 < n_iter)
            def _():
                issue(c + 2, slot)
        return _

    lax.fori_loop(0, BASE_CH // 2, pair_body, None)

    @pl.when(n_iter > BASE_CH)
    def _():
        drain_store(0)
        drain_gathers(0)
        compute(0)
        fire_store(BASE_CH, 0)

    drain_store(1)
    drain_store(0)


@functools.partial(
    pl.kernel,
    mesh=_mesh,
    out_type=(
        jax.ShapeDtypeStruct((N, D_EDGE), jnp.float32),
        jax.ShapeDtypeStruct((N, OUT), jnp.float32),
    ),
    scratch_types=[
        pltpu.VMEM((MAXC * KN * S,), jnp.int32),
        pltpu.VMEM((2, KN * S, D_EDGE), jnp.float32),
        pltpu.VMEM((2, KN * S, OUT), jnp.float32),
        pltpu.VMEM((2, KN * S, OUT), jnp.float32),
        pltpu.VMEM((2, KN, D_EDGE), jnp.float32),
        pltpu.VMEM((2, KN, OUT), jnp.float32),
        pltpu.SemaphoreType.DMA,
        pltpu.SemaphoreType.DMA,
    ],
    compiler_params=_sc_params,
)
def _sc_node_stage(n2e_hbm, ee_hbm, s_hbm, t_hbm, em_hbm, me_hbm,
                   idx_v, er_v, nr_v, tr_v, em_v, me_v, sem_g, sem_o):
    wid, base_chunk, n_iter = _chunk_range()
    i_start = base_chunk * KN * S
    CL = KN * S

    pltpu.sync_copy(n2e_hbm.at[pl.ds(i_start, BASE_CH * CL)],
                    idx_v.at[pl.ds(0, BASE_CH * CL)])

    @pl.when(wid < EXTRA)
    def _():
        pltpu.sync_copy(n2e_hbm.at[pl.ds(i_start + BASE_CH * CL, CL)],
                        idx_v.at[pl.ds(BASE_CH * CL, CL)])

    def issue(c, slot):
        pltpu.async_copy(ee_hbm.at[idx_v.at[pl.ds(c * CL, CL)]], er_v.at[slot], sem_g)
        pltpu.async_copy(s_hbm.at[idx_v.at[pl.ds(c * CL, CL)]], nr_v.at[slot], sem_g)
        pltpu.async_copy(t_hbm.at[idx_v.at[pl.ds(c * CL, CL)]], tr_v.at[slot], sem_g)

    def drain_gathers(slot):
        pltpu.make_async_copy(ee_hbm.at[idx_v.at[pl.ds(0, CL)]], er_v.at[slot], sem_g).wait()
        pltpu.make_async_copy(s_hbm.at[idx_v.at[pl.ds(0, CL)]], nr_v.at[slot], sem_g).wait()
        pltpu.make_async_copy(t_hbm.at[idx_v.at[pl.ds(0, CL)]], tr_v.at[slot], sem_g).wait()

    def compute(slot):
        for jn in range(KN):
            def acc_body(s, accs):
                base = jn * S + s
                new = tuple(
                    accs[k] + jnp.maximum(
                        nr_v[slot, base, pl.ds(k * L, L)]
                        + tr_v[slot, base, pl.ds(k * L, L)], 0.0)
                    for k in range(OUT // L)
                ) + (accs[OUT // L] + er_v[slot, base, pl.ds(0, D_EDGE)],)
                return new

            zero64 = tuple(jnp.zeros((L,), jnp.float32) for _ in range(OUT // L))
            zero16 = (jnp.zeros((D_EDGE,), jnp.float32),)
            accs = lax.fori_loop(0, S, acc_body, zero64 + zero16)
            for k in range(OUT // L):
                me_v[slot, jn, pl.ds(k * L, L)] = accs[k]
            em_v[slot, jn, pl.ds(0, D_EDGE)] = accs[OUT // L]

    def fire_store(c, slot):
        n0 = (base_chunk + c) * KN
        pltpu.async_copy(em_v.at[slot], em_hbm.at[pl.ds(n0, KN)], sem_o)
        pltpu.async_copy(me_v.at[slot], me_hbm.at[pl.ds(n0, KN)], sem_o)

    def drain_store(slot):
        pltpu.make_async_copy(em_v.at[slot], em_hbm.at[pl.ds(0, KN)], sem_o).wait()
        pltpu.make_async_copy(me_v.at[slot], me_hbm.at[pl.ds(0, KN)], sem_o).wait()

    issue(0, 0)
    issue(1, 1)

    def pair_body(p, _):
        for slot in range(2):
            c = 2 * p + slot

            @pl.when(c >= 2)
            def _():
                drain_store(slot)

            drain_gathers(slot)
            compute(slot)
            fire_store(c, slot)

            @pl.when(c + 2 < n_iter)
            def _():
                issue(c + 2, slot)
        return _

    lax.fori_loop(0, BASE_CH // 2, pair_body, None)

    @pl.when(n_iter > BASE_CH)
    def _():
        drain_store(0)
        drain_gathers(0)
        compute(0)
        fire_store(BASE_CH, 0)

    drain_store(1)
    drain_store(0)


def _prep_body(feats_ref, wp_ref, wen0_ref, wep_ref, wes0_ref, wnn0_ref,
               g_ref, wces_ref, wcn0_ref):
    h = jnp.dot(feats_ref[...], wp_ref[...], preferred_element_type=jnp.float32)
    g_ref[...] = 0.5 * jnp.dot(h, wen0_ref[...], preferred_element_type=jnp.float32)
    wces_ref[...] = jnp.dot(wep_ref[...], wes0_ref[...], preferred_element_type=jnp.float32)
    wcn0_ref[...] = jnp.dot(wep_ref[...], wnn0_ref[...], preferred_element_type=jnp.float32)


def _t_body(ee_ref, w_ref, t_ref):
    t_ref[...] = jnp.dot(ee_ref[...], w_ref[...], preferred_element_type=jnp.float32)


def _final_body(id_ref, em_ref, me_ref, wns0_ref, wcn0_ref, wns1_ref, wnn1_ref,
                o_ref):
    inv_s = 1.0 / S
    m0 = jnp.dot(em_ref[...] * inv_s, wcn0_ref[...], preferred_element_type=jnp.float32)
    nf0 = jnp.maximum(jnp.dot(id_ref[...], wns0_ref[...], preferred_element_type=jnp.float32) + m0, 0.0)
    m1 = jnp.dot(me_ref[...] * inv_s, wnn1_ref[...], preferred_element_type=jnp.float32)
    nf1 = jnp.maximum(jnp.dot(nf0, wns1_ref[...], preferred_element_type=jnp.float32) + m1, 0.0)
    o_ref[...] = jnp.concatenate([nf0, nf1], axis=-1)


_TBLOCK = 3200


def kernel(feats, node2edge_idx, edge_emb, edge_node_adj, id_emb,
           W_prep, W_edge_prep,
           W_e_self_0, W_e_neigh_0, W_n_self_0, W_n_neigh_0,
           W_e_self_1, W_e_neigh_1, W_n_self_1, W_n_neigh_1):
    g, wces, wcn0 = pl.pallas_call(
        _prep_body,
        out_shape=[
            jax.ShapeDtypeStruct((N, P), jnp.float32),
            jax.ShapeDtypeStruct((D_EDGE, OUT), jnp.float32),
            jax.ShapeDtypeStruct((D_EDGE, OUT), jnp.float32),
        ],
    )(feats, W_prep, W_e_neigh_0, W_edge_prep, W_e_self_0, W_n_neigh_0)

    a0 = jnp.reshape(edge_node_adj[:, 0], (E,))
    a1 = jnp.reshape(edge_node_adj[:, 1], (E,))

    t = pl.pallas_call(
        _t_body,
        grid=(E // _TBLOCK,),
        in_specs=[
            pl.BlockSpec((_TBLOCK, D_EDGE), lambda i: (i, 0)),
            pl.BlockSpec((D_EDGE, OUT), lambda i: (0, 0)),
        ],
        out_specs=pl.BlockSpec((_TBLOCK, OUT), lambda i: (i, 0)),
        out_shape=jax.ShapeDtypeStruct((E, OUT), jnp.float32),
    )(edge_emb, wces)

    s = _sc_edge_stage(g, a0, a1)

    n2e_flat = jnp.reshape(node2edge_idx, (N * S,))
    em_sum, me_sum = _sc_node_stage(n2e_flat, edge_emb, s, t)

    out = pl.pallas_call(
        _final_body,
        out_shape=jax.ShapeDtypeStruct((N, 2 * OUT), jnp.float32),
    )(id_emb, em_sum, me_sum, W_n_self_0, wcn0, W_n_self_1, W_n_neigh_1)

    return out[None]


# speedup vs baseline: 11.6201x; 1.4536x over previous
import functools

import jax
import jax.numpy as jnp
from jax import lax
from jax.experimental import pallas as pl
from jax.experimental.pallas import tpu as pltpu
from jax.experimental.pallas import tpu_sc as plsc

N = 10000
S = 32
E = 320000
D_FEAT = 128
D_EDGE = 16
P = 64
OUT = 64

NC = 2
NS = 16
NW = NC * NS
L = 16

EC = 128
N_ECHUNK = E // EC
KN = 4
N_NCHUNK = N // KN
BASE_CH = N_ECHUNK // NW
EXTRA = N_ECHUNK - BASE_CH * NW
MAXC = BASE_CH + 1

_mesh = plsc.VectorSubcoreMesh(core_axis_name="c", subcore_axis_name="s")
_sc_params = pltpu.CompilerParams(use_tc_tiling_on_sc=False)


def _worker_id():
    return lax.axis_index("s") * NC + lax.axis_index("c")


def _chunk_range():
    wid = _worker_id()
    base_chunk = wid * BASE_CH + jnp.minimum(wid, EXTRA)
    n_iter = BASE_CH + jnp.where(wid < EXTRA, 1, 0)
    return wid, base_chunk, n_iter


@functools.partial(
    pl.kernel,
    mesh=_mesh,
    out_type=jax.ShapeDtypeStruct((E, OUT), jnp.float32),
    scratch_types=[
        pltpu.VMEM((MAXC * EC,), jnp.int32),
        pltpu.VMEM((MAXC * EC,), jnp.int32),
        pltpu.VMEM((2, EC, OUT), jnp.float32),
        pltpu.VMEM((2, EC, OUT), jnp.float32),
        pltpu.VMEM((2, EC, OUT), jnp.float32),
        pltpu.SemaphoreType.DMA,
        pltpu.SemaphoreType.DMA,
    ],
    compiler_params=_sc_params,
)
def _sc_edge_stage(g_hbm, a0_hbm, a1_hbm, s_hbm,
                   i0_v, i1_v, r0_v, r1_v, o_v, sem_g, sem_o):
    wid, base_chunk, n_iter = _chunk_range()
    e_start = base_chunk * EC

    pltpu.sync_copy(a0_hbm.at[pl.ds(e_start, BASE_CH * EC)],
                    i0_v.at[pl.ds(0, BASE_CH * EC)])
    pltpu.sync_copy(a1_hbm.at[pl.ds(e_start, BASE_CH * EC)],
                    i1_v.at[pl.ds(0, BASE_CH * EC)])

    @pl.when(wid < EXTRA)
    def _():
        pltpu.sync_copy(a0_hbm.at[pl.ds(e_start + BASE_CH * EC, EC)],
                        i0_v.at[pl.ds(BASE_CH * EC, EC)])
        pltpu.sync_copy(a1_hbm.at[pl.ds(e_start + BASE_CH * EC, EC)],
                        i1_v.at[pl.ds(BASE_CH * EC, EC)])

    def issue(c, slot):
        pltpu.async_copy(g_hbm.at[i0_v.at[pl.ds(c * EC, EC)]], r0_v.at[slot], sem_g)
        pltpu.async_copy(g_hbm.at[i1_v.at[pl.ds(c * EC, EC)]], r1_v.at[slot], sem_g)

    def drain_gathers(slot):
        pltpu.make_async_copy(g_hbm.at[i0_v.at[pl.ds(0, EC)]], r0_v.at[slot], sem_g).wait()
        pltpu.make_async_copy(g_hbm.at[i1_v.at[pl.ds(0, EC)]], r1_v.at[slot], sem_g).wait()

    def compute(slot):
        def body(c2, _):
            for u in range(2):
                c = c2 * 2 + u
                for k in range(OUT // L):
                    sl = pl.ds(k * L, L)
                    o_v[slot, c, sl] = r0_v[slot, c, sl] + r1_v[slot, c, sl]
            return _
        lax.fori_loop(0, EC // 2, body, None)

    def fire_store(c, slot):
        pltpu.async_copy(o_v.at[slot], s_hbm.at[pl.ds((base_chunk + c) * EC, EC)], sem_o)

    def drain_store(slot):
        pltpu.make_async_copy(o_v.at[slot], s_hbm.at[pl.ds(0, EC)], sem_o).wait()

    issue(0, 0)
    issue(1, 1)

    def pair_body(p, _):
        for slot in range(2):
            c = 2 * p + slot

            @pl.when(c >= 2)
            def _():
                drain_store(slot)

            drain_gathers(slot)
            compute(slot)
            fire_store(c, slot)

            @pl.when(c + 2 < n_iter)
            def _():
                issue(c + 2, slot)
        return _

    lax.fori_loop(0, BASE_CH // 2, pair_body, None)

    @pl.when(n_iter > BASE_CH)
    def _():
        drain_store(0)
        drain_gathers(0)
        compute(0)
        fire_store(BASE_CH, 0)

    drain_store(1)
    drain_store(0)


@functools.partial(
    pl.kernel,
    mesh=_mesh,
    out_type=(
        jax.ShapeDtypeStruct((N, OUT), jnp.float32),
        jax.ShapeDtypeStruct((N, OUT), jnp.float32),
    ),
    scratch_types=[
        pltpu.VMEM((MAXC * KN * S,), jnp.int32),
        pltpu.VMEM((2, KN * S, OUT), jnp.float32),
        pltpu.VMEM((2, KN * S, 2 * OUT), jnp.float32),
        pltpu.VMEM((2, KN, OUT), jnp.float32),
        pltpu.VMEM((2, KN, OUT), jnp.float32),
        pltpu.SemaphoreType.DMA,
        pltpu.SemaphoreType.DMA,
    ],
    compiler_params=_sc_params,
)
def _sc_node_stage(n2e_hbm, s_hbm, tq_hbm, m0_hbm, me_hbm,
                   idx_v, nr_v, tr_v, m0_v, me_v, sem_g, sem_o):
    wid, base_chunk, n_iter = _chunk_range()
    i_start = base_chunk * KN * S
    CL = KN * S

    pltpu.sync_copy(n2e_hbm.at[pl.ds(i_start, BASE_CH * CL)],
                    idx_v.at[pl.ds(0, BASE_CH * CL)])

    @pl.when(wid < EXTRA)
    def _():
        pltpu.sync_copy(n2e_hbm.at[pl.ds(i_start + BASE_CH * CL, CL)],
                        idx_v.at[pl.ds(BASE_CH * CL, CL)])

    def issue(c, slot):
        pltpu.async_copy(s_hbm.at[idx_v.at[pl.ds(c * CL, CL)]], nr_v.at[slot], sem_g)
        pltpu.async_copy(tq_hbm.at[idx_v.at[pl.ds(c * CL, CL)]], tr_v.at[slot], sem_g)

    def drain_gathers(slot):
        pltpu.make_async_copy(s_hbm.at[idx_v.at[pl.ds(0, CL)]], nr_v.at[slot], sem_g).wait()
        pltpu.make_async_copy(tq_hbm.at[idx_v.at[pl.ds(0, CL)]], tr_v.at[slot], sem_g).wait()

    def compute(slot):
        for jn in range(KN):
            def acc_body(s, accs):
                base = jn * S + s
                new = tuple(
                    accs[k] + jnp.maximum(
                        nr_v[slot, base, pl.ds(k * L, L)]
                        + tr_v[slot, base, pl.ds(k * L, L)], 0.0)
                    for k in range(OUT // L)
                ) + tuple(
                    accs[OUT // L + k] + tr_v[slot, base, pl.ds(OUT + k * L, L)]
                    for k in range(OUT // L)
                )
                return new

            zeros = tuple(jnp.zeros((L,), jnp.float32) for _ in range(2 * (OUT // L)))
            accs = lax.fori_loop(0, S, acc_body, zeros)
            for k in range(OUT // L):
                me_v[slot, jn, pl.ds(k * L, L)] = accs[k]
                m0_v[slot, jn, pl.ds(k * L, L)] = accs[OUT // L + k]

    def fire_store(c, slot):
        n0 = (base_chunk + c) * KN
        pltpu.async_copy(m0_v.at[slot], m0_hbm.at[pl.ds(n0, KN)], sem_o)
        pltpu.async_copy(me_v.at[slot], me_hbm.at[pl.ds(n0, KN)], sem_o)

    def drain_store(slot):
        pltpu.make_async_copy(m0_v.at[slot], m0_hbm.at[pl.ds(0, KN)], sem_o).wait()
        pltpu.make_async_copy(me_v.at[slot], me_hbm.at[pl.ds(0, KN)], sem_o).wait()

    issue(0, 0)
    issue(1, 1)

    def pair_body(p, _):
        for slot in range(2):
            c = 2 * p + slot

            @pl.when(c >= 2)
            def _():
                drain_store(slot)

            drain_gathers(slot)
            compute(slot)
            fire_store(c, slot)

            @pl.when(c + 2 < n_iter)
            def _():
                issue(c + 2, slot)
        return _

    lax.fori_loop(0, BASE_CH // 2, pair_body, None)

    @pl.when(n_iter > BASE_CH)
    def _():
        drain_store(0)
        drain_gathers(0)
        compute(0)
        fire_store(BASE_CH, 0)

    drain_store(1)
    drain_store(0)


def _prep_body(feats_ref, wp_ref, wen0_ref, wep_ref, wes0_ref, wnn0_ref,
               g_ref, w2_ref):
    h = jnp.dot(feats_ref[...], wp_ref[...], preferred_element_type=jnp.float32)
    g_ref[...] = 0.5 * jnp.dot(h, wen0_ref[...], preferred_element_type=jnp.float32)
    wces = jnp.dot(wep_ref[...], wes0_ref[...], preferred_element_type=jnp.float32)
    wcn0 = jnp.dot(wep_ref[...], wnn0_ref[...], preferred_element_type=jnp.float32)
    w2_ref[...] = jnp.concatenate([wces, wcn0], axis=-1)


def _tq_body(ee_ref, w_ref, tq_ref):
    tq_ref[...] = jnp.dot(ee_ref[...], w_ref[...], preferred_element_type=jnp.float32)


def _final_body(id_ref, m0_ref, me_ref, wns0_ref, wns1_ref, wnn1_ref,
                o_ref):
    inv_s = 1.0 / S
    nf0 = jnp.maximum(jnp.dot(id_ref[...], wns0_ref[...], preferred_element_type=jnp.float32)
                      + m0_ref[...] * inv_s, 0.0)
    m1 = jnp.dot(me_ref[...] * inv_s, wnn1_ref[...], preferred_element_type=jnp.float32)
    nf1 = jnp.maximum(jnp.dot(nf0, wns1_ref[...], preferred_element_type=jnp.float32) + m1, 0.0)
    o_ref[...] = jnp.concatenate([nf0, nf1], axis=-1)


_TBLOCK = 3200


def kernel(feats, node2edge_idx, edge_emb, edge_node_adj, id_emb,
           W_prep, W_edge_prep,
           W_e_self_0, W_e_neigh_0, W_n_self_0, W_n_neigh_0,
           W_e_self_1, W_e_neigh_1, W_n_self_1, W_n_neigh_1):
    g, w2 = pl.pallas_call(
        _prep_body,
        out_shape=[
            jax.ShapeDtypeStruct((N, P), jnp.float32),
            jax.ShapeDtypeStruct((D_EDGE, 2 * OUT), jnp.float32),
        ],
    )(feats, W_prep, W_e_neigh_0, W_edge_prep, W_e_self_0, W_n_neigh_0)

    a0 = jnp.reshape(edge_node_adj[:, 0], (E,))
    a1 = jnp.reshape(edge_node_adj[:, 1], (E,))

    tq = pl.pallas_call(
        _tq_body,
        grid=(E // _TBLOCK,),
        in_specs=[
            pl.BlockSpec((_TBLOCK, D_EDGE), lambda i: (i, 0)),
            pl.BlockSpec((D_EDGE, 2 * OUT), lambda i: (0, 0)),
        ],
        out_specs=pl.BlockSpec((_TBLOCK, 2 * OUT), lambda i: (i, 0)),
        out_shape=jax.ShapeDtypeStruct((E, 2 * OUT), jnp.float32),
    )(edge_emb, w2)

    s = _sc_edge_stage(g, a0, a1)

    n2e_flat = jnp.reshape(node2edge_idx, (N * S,))
    m0_sum, me_sum = _sc_node_stage(n2e_flat, s, tq)

    out = pl.pallas_call(
        _final_body,
        out_shape=jax.ShapeDtypeStruct((N, 2 * OUT), jnp.float32),
    )(id_emb, m0_sum, me_sum, W_n_self_0, W_n_self_1, W_n_neigh_1)

    return out[None]
